# baseline (speedup 1.0000x reference)
import jax
import jax.numpy as jnp
from jax.experimental import pallas as pl
from jax.experimental.pallas import tpu as pltpu

f32 = jnp.float32
bf16 = jnp.bfloat16


def _prep_kernel(eb_ref, e_ref, x_ref, s_ref, gb_ref, ub_ref,
                 gcat_ref, ucat_ref,
                 a_ref, xa_ref, st_ref, bg_ref, bu_ref, wg_ref, wu_ref):
    eb = eb_ref[...]
    e = e_ref[...]
    g = jax.lax.dot_general(eb, e, (((1,), (1,)), ((), ())),
                            preferred_element_type=f32)
    g = jnp.maximum(g, 0.0)
    g = g - jnp.max(g, axis=1, keepdims=True)
    eg = jnp.exp(g)
    a = (eg / jnp.sum(eg, axis=1, keepdims=True)).astype(bf16)
    a_ref[...] = a
    nn, ci, bb = x_ref.shape
    x2 = x_ref[...].reshape(nn, ci * bb)
    aggx = jnp.dot(a, x2, preferred_element_type=f32).astype(bf16)
    xa_ref[:, ci:, :] = aggx.reshape(a.shape[0], ci, bb)
    nbw = a.shape[0]
    base = pl.program_id(0) * nbw
    xa_ref[:, :ci, :] = x_ref[pl.ds(base, nbw)]
    bsz = s_ref.shape[0]
    row = jax.lax.broadcasted_iota(jnp.int32, (bsz, bsz), 0)
    col = jax.lax.broadcasted_iota(jnp.int32, (bsz, bsz), 1)
    ident = jnp.where(row == col, 1.0, 0.0).astype(bf16)
    for j in range(nbw):
        sj = s_ref[:, j, :].astype(bf16)
        st_ref[j] = jax.lax.dot_general(
            sj, ident, (((0,), (0,)), ((), ())),
            preferred_element_type=f32).astype(bf16)
    bg = jnp.dot(eb, gb_ref[...], preferred_element_type=f32)
    bu = jnp.dot(eb, ub_ref[...], preferred_element_type=f32)
    bg_ref[...] = bg.reshape(bg.shape[0], bg.shape[1], 1)
    bu_ref[...] = bu.reshape(bu.shape[0], bu.shape[1], 1)
    nb, kc, og = wg_ref.shape
    ou = wu_ref.shape[2]
    wg = jnp.dot(eb, gcat_ref[...], preferred_element_type=f32).astype(bf16)
    wu = jnp.dot(eb, ucat_ref[...], preferred_element_type=f32).astype(bf16)
    wg_ref[...] = wg.reshape(nb, kc, og)
    wu_ref[...] = wu.reshape(nb, kc, ou)


def _agg_kernel(a_ref, v_ref, o_ref):
    nn, hb, bb = v_ref.shape
    v2 = v_ref[...].reshape(nn, hb * bb)
    o2 = jnp.dot(a_ref[...], v2, preferred_element_type=f32).astype(bf16)
    o_ref[...] = o2.reshape(nn, hb, bb)


def _gate_kernel(s_ref, aggs_ref, xa_ref, wg_ref, bg_ref,
                 t_ref, r_ref):
    nb = s_ref.shape[0]
    for j in range(nb):
        s = s_ref[j]
        cat = jnp.concatenate(
            [s, aggs_ref[j], xa_ref[j]], axis=0)
        pre = jax.lax.dot_general(wg_ref[j], cat, (((0,), (0,)), ((), ())),
                                  preferred_element_type=f32)
        zr = jax.nn.sigmoid(pre + bg_ref[j])
        h = s.shape[0]
        z = zr[:h, :]
        t_ref[j] = (z * s.astype(f32)).astype(bf16)
        r_ref[j] = zr[h:, :].astype(bf16)


def _cand_kernel(t_ref, aggt_ref, xa_ref, wu_ref, bu_ref,
                 r_ref, s_ref, h_ref):
    nb = t_ref.shape[0]
    for j in range(nb):
        cat = jnp.concatenate(
            [t_ref[j], aggt_ref[j], xa_ref[j]], axis=0)
        pre = jax.lax.dot_general(wu_ref[j], cat, (((0,), (0,)), ((), ())),
                                  preferred_element_type=f32)
        hc = jnp.tanh(pre + bu_ref[j])
        r = r_ref[j].astype(f32)
        hv = r * s_ref[j].astype(f32) + (1.0 - r) * hc
        h_ref[j] = hv.astype(bf16)


def kernel(x, state, node_emb, gate_w, gate_b, upd_w, upd_b):
    b, n, ci = x.shape
    h = state.shape[-1]
    d = node_emb.shape[-1]
    out_dtype = state.dtype
    kc = 2 * h + 2 * ci

    e = node_emb.astype(f32)
    x_t = x.astype(bf16).transpose(1, 2, 0)
    s_f = state.astype(f32)

    gw = gate_w.astype(f32)
    uw = upd_w.astype(f32)
    gcat = jnp.concatenate(
        [gw[:, 0, ci:, :], gw[:, 1, ci:, :],
         gw[:, 0, :ci, :], gw[:, 1, :ci, :]], axis=1).reshape(d, kc * 2 * h)
    ucat = jnp.concatenate(
        [uw[:, 0, ci:, :], uw[:, 1, ci:, :],
         uw[:, 0, :ci, :], uw[:, 1, :ci, :]], axis=1).reshape(d, kc * h)

    nb_w = 64 if n % 64 == 0 else n
    a_adj, xa3, s_t, bg3, bu3, wg3, wu3 = pl.pallas_call(
        _prep_kernel,
        grid=(n // nb_w,),
        out_shape=(jax.ShapeDtypeStruct((n, n), bf16),
                   jax.ShapeDtypeStruct((n, 2 * ci, b), bf16),
                   jax.ShapeDtypeStruct((n, h, b), bf16),
                   jax.ShapeDtypeStruct((n, 2 * h, 1), f32),
                   jax.ShapeDtypeStruct((n, h, 1), f32),
                   jax.ShapeDtypeStruct((n, kc, 2 * h), bf16),
                   jax.ShapeDtypeStruct((n, kc, h), bf16)),
        in_specs=[
            pl.BlockSpec((nb_w, d), lambda i: (i, 0)),
            pl.BlockSpec((n, d), lambda i: (0, 0)),
            pl.BlockSpec((n, ci, b), lambda i: (0, 0, 0)),
            pl.BlockSpec((b, nb_w, h), lambda i: (0, i, 0)),
            pl.BlockSpec((d, 2 * h), lambda i: (0, 0)),
            pl.BlockSpec((d, h), lambda i: (0, 0)),
            pl.BlockSpec((d, kc * 2 * h), lambda i: (0, 0)),
            pl.BlockSpec((d, kc * h), lambda i: (0, 0)),
        ],
        out_specs=(pl.BlockSpec((nb_w, n), lambda i: (i, 0)),
                   pl.BlockSpec((nb_w, 2 * ci, b), lambda i: (i, 0, 0)),
                   pl.BlockSpec((nb_w, h, b), lambda i: (i, 0, 0)),
                   pl.BlockSpec((nb_w, 2 * h, 1), lambda i: (i, 0, 0)),
                   pl.BlockSpec((nb_w, h, 1), lambda i: (i, 0, 0)),
                   pl.BlockSpec((nb_w, kc, 2 * h), lambda i: (i, 0, 0)),
                   pl.BlockSpec((nb_w, kc, h), lambda i: (i, 0, 0))),
        compiler_params=pltpu.CompilerParams(
            dimension_semantics=("parallel",)),
    )(e, e, x_t, s_f, gate_b.astype(f32), upd_b.astype(f32), gcat, ucat)

    def agg(v3):
        feat = v3.shape[1]
        fb = 16 if feat % 16 == 0 else feat
        return pl.pallas_call(
            _agg_kernel,
            grid=(feat // fb,),
            out_shape=jax.ShapeDtypeStruct((n, feat, b), bf16),
            in_specs=[
                pl.BlockSpec((n, n), lambda i: (0, 0)),
                pl.BlockSpec((n, fb, b), lambda i: (0, i, 0)),
            ],
            out_specs=pl.BlockSpec((n, fb, b), lambda i: (0, i, 0)),
            compiler_params=pltpu.CompilerParams(
                dimension_semantics=("parallel",)),
        )(a_adj, v3)

    aggs3 = agg(s_t)

    nb_g = 64 if n % 64 == 0 else n
    t3, r3 = pl.pallas_call(
        _gate_kernel,
        grid=(n // nb_g,),
        out_shape=(jax.ShapeDtypeStruct((n, h, b), bf16),
                   jax.ShapeDtypeStruct((n, h, b), bf16)),
        in_specs=[
            pl.BlockSpec((nb_g, h, b), lambda i: (i, 0, 0)),
            pl.BlockSpec((nb_g, h, b), lambda i: (i, 0, 0)),
            pl.BlockSpec((nb_g, 2 * ci, b), lambda i: (i, 0, 0)),
            pl.BlockSpec((nb_g, kc, 2 * h), lambda i: (i, 0, 0)),
            pl.BlockSpec((nb_g, 2 * h, 1), lambda i: (i, 0, 0)),
        ],
        out_specs=(pl.BlockSpec((nb_g, h, b), lambda i: (i, 0, 0)),
                   pl.BlockSpec((nb_g, h, b), lambda i: (i, 0, 0))),
        compiler_params=pltpu.CompilerParams(
            dimension_semantics=("parallel",)),
    )(s_t, aggs3, xa3, wg3, bg3)

    aggt3 = agg(t3)

    h3 = pl.pallas_call(
        _cand_kernel,
        grid=(n // nb_g,),
        out_shape=jax.ShapeDtypeStruct((n, h, b), bf16),
        in_specs=[
            pl.BlockSpec((nb_g, h, b), lambda i: (i, 0, 0)),
            pl.BlockSpec((nb_g, h, b), lambda i: (i, 0, 0)),
            pl.BlockSpec((nb_g, 2 * ci, b), lambda i: (i, 0, 0)),
            pl.BlockSpec((nb_g, kc, h), lambda i: (i, 0, 0)),
            pl.BlockSpec((nb_g, h, 1), lambda i: (i, 0, 0)),
            pl.BlockSpec((nb_g, h, b), lambda i: (i, 0, 0)),
            pl.BlockSpec((nb_g, h, b), lambda i: (i, 0, 0)),
        ],
        out_specs=pl.BlockSpec((nb_g, h, b), lambda i: (i, 0, 0)),
        compiler_params=pltpu.CompilerParams(
            dimension_semantics=("parallel",)),
    )(t3, aggt3, xa3, wu3, bu3, r3, s_t)

    return h3.transpose(2, 0, 1).astype(out_dtype)


# speedup vs baseline: 1.3536x; 1.3536x over previous
import jax
import jax.numpy as jnp
from jax.experimental import pallas as pl
from jax.experimental.pallas import tpu as pltpu

f32 = jnp.float32
bf16 = jnp.bfloat16


def _prep_kernel(eb_ref, e_ref, x_ref, gb_ref, ub_ref,
                 gcat_ref, ucat_ref,
                 a_ref, xa_ref, bg_ref, bu_ref, wg_ref, wu_ref):
    eb = eb_ref[...]
    e = e_ref[...]
    g = jax.lax.dot_general(eb, e, (((1,), (1,)), ((), ())),
                            preferred_element_type=f32)
    g = jnp.maximum(g, 0.0)
    g = g - jnp.max(g, axis=1, keepdims=True)
    eg = jnp.exp(g)
    a = (eg / jnp.sum(eg, axis=1, keepdims=True)).astype(bf16)
    a_ref[...] = a
    nn, ci, bb = x_ref.shape
    x2 = x_ref[...].reshape(nn, ci * bb)
    aggx = jnp.dot(a, x2, preferred_element_type=f32).astype(bf16)
    xa_ref[:, ci:, :] = aggx.reshape(a.shape[0], ci, bb)
    nbw = a.shape[0]
    base = pl.program_id(0) * nbw
    xa_ref[:, :ci, :] = x_ref[pl.ds(base, nbw)]
    bg = jnp.dot(eb, gb_ref[...], preferred_element_type=f32)
    bu = jnp.dot(eb, ub_ref[...], preferred_element_type=f32)
    bg_ref[...] = bg.reshape(bg.shape[0], bg.shape[1], 1)
    bu_ref[...] = bu.reshape(bu.shape[0], bu.shape[1], 1)
    nb, kc, og = wg_ref.shape
    ou = wu_ref.shape[2]
    wg = jnp.dot(eb, gcat_ref[...], preferred_element_type=f32).astype(bf16)
    wu = jnp.dot(eb, ucat_ref[...], preferred_element_type=f32).astype(bf16)
    wg_ref[...] = wg.reshape(nb, kc, og)
    wu_ref[...] = wu.reshape(nb, kc, ou)


def _agg_kernel(a_ref, v_ref, o_ref):
    nn, hb, bb = v_ref.shape
    v2 = v_ref[...].reshape(nn, hb * bb)
    o2 = jnp.dot(a_ref[...], v2, preferred_element_type=f32).astype(bf16)
    o_ref[...] = o2.reshape(nn, hb, bb)


def _gate_kernel(s_ref, aggs_ref, xa_ref, wg_ref, bg_ref,
                 t_ref, r_ref):
    nb = s_ref.shape[0]
    for j in range(nb):
        s = s_ref[j]
        cat = jnp.concatenate(
            [s, aggs_ref[j], xa_ref[j]], axis=0)
        pre = jax.lax.dot_general(wg_ref[j], cat, (((0,), (0,)), ((), ())),
                                  preferred_element_type=f32)
        zr = jax.nn.sigmoid(pre + bg_ref[j])
        h = s.shape[0]
        z = zr[:h, :]
        t_ref[j] = (z * s.astype(f32)).astype(bf16)
        r_ref[j] = zr[h:, :].astype(bf16)


def _cand_kernel(t_ref, aggt_ref, xa_ref, wu_ref, bu_ref,
                 r_ref, s_ref, h_ref):
    nb = t_ref.shape[0]
    for j in range(nb):
        cat = jnp.concatenate(
            [t_ref[j], aggt_ref[j], xa_ref[j]], axis=0)
        pre = jax.lax.dot_general(wu_ref[j], cat, (((0,), (0,)), ((), ())),
                                  preferred_element_type=f32)
        hc = jnp.tanh(pre + bu_ref[j])
        r = r_ref[j].astype(f32)
        hv = r * s_ref[j].astype(f32) + (1.0 - r) * hc
        h_ref[j] = hv.astype(bf16)


def kernel(x, state, node_emb, gate_w, gate_b, upd_w, upd_b):
    b, n, ci = x.shape
    h = state.shape[-1]
    d = node_emb.shape[-1]
    out_dtype = state.dtype
    kc = 2 * h + 2 * ci

    e = node_emb.astype(f32)
    x_t = x.astype(bf16).transpose(1, 2, 0)
    s_t = state.astype(bf16).transpose(1, 2, 0)

    gw = gate_w.astype(f32)
    uw = upd_w.astype(f32)
    gcat = jnp.concatenate(
        [gw[:, 0, ci:, :], gw[:, 1, ci:, :],
         gw[:, 0, :ci, :], gw[:, 1, :ci, :]], axis=1).reshape(d, kc * 2 * h)
    ucat = jnp.concatenate(
        [uw[:, 0, ci:, :], uw[:, 1, ci:, :],
         uw[:, 0, :ci, :], uw[:, 1, :ci, :]], axis=1).reshape(d, kc * h)

    nb_w = 64 if n % 64 == 0 else n
    a_adj, xa3, bg3, bu3, wg3, wu3 = pl.pallas_call(
        _prep_kernel,
        grid=(n // nb_w,),
        out_shape=(jax.ShapeDtypeStruct((n, n), bf16),
                   jax.ShapeDtypeStruct((n, 2 * ci, b), bf16),
                   jax.ShapeDtypeStruct((n, 2 * h, 1), f32),
                   jax.ShapeDtypeStruct((n, h, 1), f32),
                   jax.ShapeDtypeStruct((n, kc, 2 * h), bf16),
                   jax.ShapeDtypeStruct((n, kc, h), bf16)),
        in_specs=[
            pl.BlockSpec((nb_w, d), lambda i: (i, 0)),
            pl.BlockSpec((n, d), lambda i: (0, 0)),
            pl.BlockSpec((n, ci, b), lambda i: (0, 0, 0)),
            pl.BlockSpec((d, 2 * h), lambda i: (0, 0)),
            pl.BlockSpec((d, h), lambda i: (0, 0)),
            pl.BlockSpec((d, kc * 2 * h), lambda i: (0, 0)),
            pl.BlockSpec((d, kc * h), lambda i: (0, 0)),
        ],
        out_specs=(pl.BlockSpec((nb_w, n), lambda i: (i, 0)),
                   pl.BlockSpec((nb_w, 2 * ci, b), lambda i: (i, 0, 0)),
                   pl.BlockSpec((nb_w, 2 * h, 1), lambda i: (i, 0, 0)),
                   pl.BlockSpec((nb_w, h, 1), lambda i: (i, 0, 0)),
                   pl.BlockSpec((nb_w, kc, 2 * h), lambda i: (i, 0, 0)),
                   pl.BlockSpec((nb_w, kc, h), lambda i: (i, 0, 0))),
        compiler_params=pltpu.CompilerParams(
            dimension_semantics=("parallel",)),
    )(e, e, x_t, gate_b.astype(f32), upd_b.astype(f32), gcat, ucat)

    def agg(v3):
        feat = v3.shape[1]
        fb = 16 if feat % 16 == 0 else feat
        return pl.pallas_call(
            _agg_kernel,
            grid=(feat // fb,),
            out_shape=jax.ShapeDtypeStruct((n, feat, b), bf16),
            in_specs=[
                pl.BlockSpec((n, n), lambda i: (0, 0)),
                pl.BlockSpec((n, fb, b), lambda i: (0, i, 0)),
            ],
            out_specs=pl.BlockSpec((n, fb, b), lambda i: (0, i, 0)),
            compiler_params=pltpu.CompilerParams(
                dimension_semantics=("parallel",)),
        )(a_adj, v3)

    aggs3 = agg(s_t)

    nb_g = 64 if n % 64 == 0 else n
    t3, r3 = pl.pallas_call(
        _gate_kernel,
        grid=(n // nb_g,),
        out_shape=(jax.ShapeDtypeStruct((n, h, b), bf16),
                   jax.ShapeDtypeStruct((n, h, b), bf16)),
        in_specs=[
            pl.BlockSpec((nb_g, h, b), lambda i: (i, 0, 0)),
            pl.BlockSpec((nb_g, h, b), lambda i: (i, 0, 0)),
            pl.BlockSpec((nb_g, 2 * ci, b), lambda i: (i, 0, 0)),
            pl.BlockSpec((nb_g, kc, 2 * h), lambda i: (i, 0, 0)),
            pl.BlockSpec((nb_g, 2 * h, 1), lambda i: (i, 0, 0)),
        ],
        out_specs=(pl.BlockSpec((nb_g, h, b), lambda i: (i, 0, 0)),
                   pl.BlockSpec((nb_g, h, b), lambda i: (i, 0, 0))),
        compiler_params=pltpu.CompilerParams(
            dimension_semantics=("parallel",)),
    )(s_t, aggs3, xa3, wg3, bg3)

    aggt3 = agg(t3)

    h3 = pl.pallas_call(
        _cand_kernel,
        grid=(n // nb_g,),
        out_shape=jax.ShapeDtypeStruct((n, h, b), bf16),
        in_specs=[
            pl.BlockSpec((nb_g, h, b), lambda i: (i, 0, 0)),
            pl.BlockSpec((nb_g, h, b), lambda i: (i, 0, 0)),
            pl.BlockSpec((nb_g, 2 * ci, b), lambda i: (i, 0, 0)),
            pl.BlockSpec((nb_g, kc, h), lambda i: (i, 0, 0)),
            pl.BlockSpec((nb_g, h, 1), lambda i: (i, 0, 0)),
            pl.BlockSpec((nb_g, h, b), lambda i: (i, 0, 0)),
            pl.BlockSpec((nb_g, h, b), lambda i: (i, 0, 0)),
        ],
        out_specs=pl.BlockSpec((nb_g, h, b), lambda i: (i, 0, 0)),
        compiler_params=pltpu.CompilerParams(
            dimension_semantics=("parallel",)),
    )(t3, aggt3, xa3, wu3, bu3, r3, s_t)

    return h3.transpose(2, 0, 1).astype(out_dtype)


# speedup vs baseline: 1.4029x; 1.0364x over previous
import jax
import jax.numpy as jnp
from jax.experimental import pallas as pl
from jax.experimental.pallas import tpu as pltpu

f32 = jnp.float32
bf16 = jnp.bfloat16


def _prep_kernel(eb_ref, e_ref, x_ref, gb_ref, ub_ref,
                 gcat_ref, ucat_ref,
                 a_ref, xa_ref, bg_ref, bu_ref, wg_ref, wu_ref):
    eb = eb_ref[...]
    e = e_ref[...]
    g = jax.lax.dot_general(eb, e, (((1,), (1,)), ((), ())),
                            preferred_element_type=f32)
    g = jnp.maximum(g, 0.0)
    g = g - jnp.max(g, axis=1, keepdims=True)
    eg = jnp.exp(g)
    a = (eg / jnp.sum(eg, axis=1, keepdims=True)).astype(bf16)
    a_ref[...] = a
    nn, ci, bb = x_ref.shape
    x2 = x_ref[...].reshape(nn, ci * bb)
    aggx = jnp.dot(a, x2, preferred_element_type=f32).astype(bf16)
    xa_ref[:, ci:, :] = aggx.reshape(a.shape[0], ci, bb)
    nbw = a.shape[0]
    base = pl.program_id(0) * nbw
    xa_ref[:, :ci, :] = x_ref[pl.ds(base, nbw)]
    bg = jnp.dot(eb, gb_ref[...], preferred_element_type=f32)
    bu = jnp.dot(eb, ub_ref[...], preferred_element_type=f32)
    bg_ref[...] = bg.reshape(bg.shape[0], bg.shape[1], 1)
    bu_ref[...] = bu.reshape(bu.shape[0], bu.shape[1], 1)
    nb, kc, og = wg_ref.shape
    ou = wu_ref.shape[2]
    wg = jnp.dot(eb, gcat_ref[...], preferred_element_type=f32).astype(bf16)
    wu = jnp.dot(eb, ucat_ref[...], preferred_element_type=f32).astype(bf16)
    wg_ref[...] = wg.reshape(nb, kc, og)
    wu_ref[...] = wu.reshape(nb, kc, ou)


def _fill_agg(a_ref, v_ref, agg_ref):
    nn, hh, bb = v_ref.shape
    a = a_ref[...]
    csz = 8 if hh % 8 == 0 else hh
    for c in range(0, hh, csz):
        vc = v_ref[:, c:c + csz, :].reshape(nn, csz * bb)
        chunk = jnp.dot(a, vc, preferred_element_type=f32).astype(bf16)
        agg_ref[:, c:c + csz, :] = chunk.reshape(nn, csz, bb)


def _gateagg_kernel(a_ref, sf_ref, xa_ref, wg_ref, bg_ref,
                    t_ref, r_ref, aggs_ref):
    nb = wg_ref.shape[0]
    i = pl.program_id(0)

    @pl.when(i == 0)
    def _():
        _fill_agg(a_ref, sf_ref, aggs_ref)

    for j in range(nb):
        jj = i * nb + j
        s = sf_ref[jj]
        cat = jnp.concatenate(
            [s, aggs_ref[jj], xa_ref[j]], axis=0)
        pre = jax.lax.dot_general(wg_ref[j], cat, (((0,), (0,)), ((), ())),
                                  preferred_element_type=f32)
        zr = jax.nn.sigmoid(pre + bg_ref[j])
        h = s.shape[0]
        z = zr[:h, :]
        t_ref[j] = (z * s.astype(f32)).astype(bf16)
        r_ref[j] = zr[h:, :].astype(bf16)


def _candagg_kernel(a_ref, tf_ref, xa_ref, wu_ref, bu_ref,
                    r_ref, s_ref, h_ref, aggt_ref):
    nb = wu_ref.shape[0]
    i = pl.program_id(0)

    @pl.when(i == 0)
    def _():
        _fill_agg(a_ref, tf_ref, aggt_ref)

    for j in range(nb):
        jj = i * nb + j
        cat = jnp.concatenate(
            [tf_ref[jj], aggt_ref[jj], xa_ref[j]], axis=0)
        pre = jax.lax.dot_general(wu_ref[j], cat, (((0,), (0,)), ((), ())),
                                  preferred_element_type=f32)
        hc = jnp.tanh(pre + bu_ref[j])
        r = r_ref[j].astype(f32)
        hv = r * s_ref[j].astype(f32) + (1.0 - r) * hc
        h_ref[j] = hv.astype(bf16)


def kernel(x, state, node_emb, gate_w, gate_b, upd_w, upd_b):
    b, n, ci = x.shape
    h = state.shape[-1]
    d = node_emb.shape[-1]
    out_dtype = state.dtype
    kc = 2 * h + 2 * ci

    e = node_emb.astype(f32)
    x_t = x.astype(bf16).transpose(1, 2, 0)
    s_t = state.astype(bf16).transpose(1, 2, 0)

    gw = gate_w.astype(f32)
    uw = upd_w.astype(f32)
    gcat = jnp.concatenate(
        [gw[:, 0, ci:, :], gw[:, 1, ci:, :],
         gw[:, 0, :ci, :], gw[:, 1, :ci, :]], axis=1).reshape(d, kc * 2 * h)
    ucat = jnp.concatenate(
        [uw[:, 0, ci:, :], uw[:, 1, ci:, :],
         uw[:, 0, :ci, :], uw[:, 1, :ci, :]], axis=1).reshape(d, kc * h)

    nb_w = 64 if n % 64 == 0 else n
    a_adj, xa3, bg3, bu3, wg3, wu3 = pl.pallas_call(
        _prep_kernel,
        grid=(n // nb_w,),
        out_shape=(jax.ShapeDtypeStruct((n, n), bf16),
                   jax.ShapeDtypeStruct((n, 2 * ci, b), bf16),
                   jax.ShapeDtypeStruct((n, 2 * h, 1), f32),
                   jax.ShapeDtypeStruct((n, h, 1), f32),
                   jax.ShapeDtypeStruct((n, kc, 2 * h), bf16),
                   jax.ShapeDtypeStruct((n, kc, h), bf16)),
        in_specs=[
            pl.BlockSpec((nb_w, d), lambda i: (i, 0)),
            pl.BlockSpec((n, d), lambda i: (0, 0)),
            pl.BlockSpec((n, ci, b), lambda i: (0, 0, 0)),
            pl.BlockSpec((d, 2 * h), lambda i: (0, 0)),
            pl.BlockSpec((d, h), lambda i: (0, 0)),
            pl.BlockSpec((d, kc * 2 * h), lambda i: (0, 0)),
            pl.BlockSpec((d, kc * h), lambda i: (0, 0)),
        ],
        out_specs=(pl.BlockSpec((nb_w, n), lambda i: (i, 0)),
                   pl.BlockSpec((nb_w, 2 * ci, b), lambda i: (i, 0, 0)),
                   pl.BlockSpec((nb_w, 2 * h, 1), lambda i: (i, 0, 0)),
                   pl.BlockSpec((nb_w, h, 1), lambda i: (i, 0, 0)),
                   pl.BlockSpec((nb_w, kc, 2 * h), lambda i: (i, 0, 0)),
                   pl.BlockSpec((nb_w, kc, h), lambda i: (i, 0, 0))),
        compiler_params=pltpu.CompilerParams(
            dimension_semantics=("parallel",)),
    )(e, e, x_t, gate_b.astype(f32), upd_b.astype(f32), gcat, ucat)

    nb_g = 32 if n % 32 == 0 else n
    cparams = pltpu.CompilerParams(
        dimension_semantics=("arbitrary",),
        vmem_limit_bytes=56 * 1024 * 1024)
    t3, r3 = pl.pallas_call(
        _gateagg_kernel,
        grid=(n // nb_g,),
        out_shape=(jax.ShapeDtypeStruct((n, h, b), bf16),
                   jax.ShapeDtypeStruct((n, h, b), bf16)),
        in_specs=[
            pl.BlockSpec((n, n), lambda i: (0, 0)),
            pl.BlockSpec((n, h, b), lambda i: (0, 0, 0)),
            pl.BlockSpec((nb_g, 2 * ci, b), lambda i: (i, 0, 0)),
            pl.BlockSpec((nb_g, kc, 2 * h), lambda i: (i, 0, 0)),
            pl.BlockSpec((nb_g, 2 * h, 1), lambda i: (i, 0, 0)),
        ],
        out_specs=(pl.BlockSpec((nb_g, h, b), lambda i: (i, 0, 0)),
                   pl.BlockSpec((nb_g, h, b), lambda i: (i, 0, 0))),
        scratch_shapes=[pltpu.VMEM((n, h, b), bf16)],
        compiler_params=cparams,
    )(a_adj, s_t, xa3, wg3, bg3)

    h3 = pl.pallas_call(
        _candagg_kernel,
        grid=(n // nb_g,),
        out_shape=jax.ShapeDtypeStruct((n, h, b), bf16),
        in_specs=[
            pl.BlockSpec((n, n), lambda i: (0, 0)),
            pl.BlockSpec((n, h, b), lambda i: (0, 0, 0)),
            pl.BlockSpec((nb_g, 2 * ci, b), lambda i: (i, 0, 0)),
            pl.BlockSpec((nb_g, kc, h), lambda i: (i, 0, 0)),
            pl.BlockSpec((nb_g, h, 1), lambda i: (i, 0, 0)),
            pl.BlockSpec((nb_g, h, b), lambda i: (i, 0, 0)),
            pl.BlockSpec((nb_g, h, b), lambda i: (i, 0, 0)),
        ],
        out_specs=pl.BlockSpec((nb_g, h, b), lambda i: (i, 0, 0)),
        scratch_shapes=[pltpu.VMEM((n, h, b), bf16)],
        compiler_params=cparams,
    )(a_adj, t3, xa3, wu3, bu3, r3, s_t)

    return h3.transpose(2, 0, 1).astype(out_dtype)


# speedup vs baseline: 1.5472x; 1.1029x over previous
import jax
import jax.numpy as jnp
from jax.experimental import pallas as pl
from jax.experimental.pallas import tpu as pltpu

f32 = jnp.float32
bf16 = jnp.bfloat16


def _prep_kernel(eb_ref, e_ref, x_ref,
                 gcat_ref, ucat_ref,
                 a_ref, xa_ref, wg_ref, wu_ref):
    eb = eb_ref[...]
    e = e_ref[...]
    g = jax.lax.dot_general(eb, e, (((1,), (1,)), ((), ())),
                            preferred_element_type=f32)
    g = jnp.maximum(g, 0.0)
    g = g - jnp.max(g, axis=1, keepdims=True)
    eg = jnp.exp(g)
    a = (eg / jnp.sum(eg, axis=1, keepdims=True)).astype(bf16)
    a_ref[...] = a
    nn, ci, bb = x_ref.shape
    x2 = x_ref[...].reshape(nn, ci * bb)
    aggx = jnp.dot(a, x2, preferred_element_type=f32).astype(bf16)
    xa_ref[:, ci:, :] = aggx.reshape(a.shape[0], ci, bb)
    nbw = a.shape[0]
    base = pl.program_id(0) * nbw
    xa_ref[:, :ci, :] = x_ref[pl.ds(base, nbw)]
    nb, kc, og = wg_ref.shape
    ou = wu_ref.shape[2]
    wg = jnp.dot(eb, gcat_ref[...], preferred_element_type=f32).astype(bf16)
    wu = jnp.dot(eb, ucat_ref[...], preferred_element_type=f32).astype(bf16)
    wg_ref[...] = wg.reshape(nb, kc, og)
    wu_ref[...] = wu.reshape(nb, kc, ou)


def _fill_agg(a_ref, v_ref, agg_ref):
    nn, hh, bb = v_ref.shape
    a = a_ref[...]
    csz = 8 if hh % 8 == 0 else hh
    for c in range(0, hh, csz):
        vc = v_ref[:, c:c + csz, :].reshape(nn, csz * bb)
        chunk = jnp.dot(a, vc, preferred_element_type=f32).astype(bf16)
        agg_ref[:, c:c + csz, :] = chunk.reshape(nn, csz, bb)


def _gateagg_kernel(a_ref, sf_ref, xa_ref, wg_ref,
                    t_ref, r_ref, aggs_ref):
    nb = wg_ref.shape[0]
    i = pl.program_id(0)

    @pl.when(i == 0)
    def _():
        _fill_agg(a_ref, sf_ref, aggs_ref)

    ones = jnp.ones((1, sf_ref.shape[2]), bf16)
    for j in range(nb):
        jj = i * nb + j
        s = sf_ref[jj]
        cat = jnp.concatenate(
            [s, aggs_ref[jj], xa_ref[j], ones], axis=0)
        pre = jax.lax.dot_general(wg_ref[j], cat, (((0,), (0,)), ((), ())),
                                  preferred_element_type=f32)
        zr = jax.nn.sigmoid(pre)
        h = s.shape[0]
        z = zr[:h, :]
        t_ref[j] = (z * s.astype(f32)).astype(bf16)
        r_ref[j] = zr[h:, :].astype(bf16)


def _candagg_kernel(a_ref, tf_ref, xa_ref, wu_ref,
                    r_ref, s_ref, h_ref, aggt_ref):
    nb = wu_ref.shape[0]
    i = pl.program_id(0)

    @pl.when(i == 0)
    def _():
        _fill_agg(a_ref, tf_ref, aggt_ref)

    ones = jnp.ones((1, tf_ref.shape[2]), bf16)
    for j in range(nb):
        jj = i * nb + j
        cat = jnp.concatenate(
            [tf_ref[jj], aggt_ref[jj], xa_ref[j], ones], axis=0)
        pre = jax.lax.dot_general(wu_ref[j], cat, (((0,), (0,)), ((), ())),
                                  preferred_element_type=f32)
        hc = jnp.tanh(pre)
        r = r_ref[j].astype(f32)
        hv = r * s_ref[j].astype(f32) + (1.0 - r) * hc
        h_ref[j] = hv.astype(bf16)


def kernel(x, state, node_emb, gate_w, gate_b, upd_w, upd_b):
    b, n, ci = x.shape
    h = state.shape[-1]
    d = node_emb.shape[-1]
    out_dtype = state.dtype
    kc = 2 * h + 2 * ci + 1

    e = node_emb.astype(f32)
    x_t = x.astype(bf16).transpose(1, 2, 0)
    s_t = state.astype(bf16).transpose(1, 2, 0)

    gw = gate_w.astype(f32)
    uw = upd_w.astype(f32)
    gcat = jnp.concatenate(
        [gw[:, 0, ci:, :], gw[:, 1, ci:, :],
         gw[:, 0, :ci, :], gw[:, 1, :ci, :],
         gate_b.astype(f32)[:, None, :]], axis=1).reshape(d, kc * 2 * h)
    ucat = jnp.concatenate(
        [uw[:, 0, ci:, :], uw[:, 1, ci:, :],
         uw[:, 0, :ci, :], uw[:, 1, :ci, :],
         upd_b.astype(f32)[:, None, :]], axis=1).reshape(d, kc * h)

    nb_w = 64 if n % 64 == 0 else n
    a_adj, xa3, wg3, wu3 = pl.pallas_call(
        _prep_kernel,
        grid=(n // nb_w,),
        out_shape=(jax.ShapeDtypeStruct((n, n), bf16),
                   jax.ShapeDtypeStruct((n, 2 * ci, b), bf16),
                   jax.ShapeDtypeStruct((n, kc, 2 * h), bf16),
                   jax.ShapeDtypeStruct((n, kc, h), bf16)),
        in_specs=[
            pl.BlockSpec((nb_w, d), lambda i: (i, 0)),
            pl.BlockSpec((n, d), lambda i: (0, 0)),
            pl.BlockSpec((n, ci, b), lambda i: (0, 0, 0)),
            pl.BlockSpec((d, kc * 2 * h), lambda i: (0, 0)),
            pl.BlockSpec((d, kc * h), lambda i: (0, 0)),
        ],
        out_specs=(pl.BlockSpec((nb_w, n), lambda i: (i, 0)),
                   pl.BlockSpec((nb_w, 2 * ci, b), lambda i: (i, 0, 0)),
                   pl.BlockSpec((nb_w, kc, 2 * h), lambda i: (i, 0, 0)),
                   pl.BlockSpec((nb_w, kc, h), lambda i: (i, 0, 0))),
        compiler_params=pltpu.CompilerParams(
            dimension_semantics=("parallel",)),
    )(e, e, x_t, gcat, ucat)

    nb_g = 32 if n % 32 == 0 else n
    cparams = pltpu.CompilerParams(
        dimension_semantics=("arbitrary",),
        vmem_limit_bytes=56 * 1024 * 1024)
    t3, r3 = pl.pallas_call(
        _gateagg_kernel,
        grid=(n // nb_g,),
        out_shape=(jax.ShapeDtypeStruct((n, h, b), bf16),
                   jax.ShapeDtypeStruct((n, h, b), bf16)),
        in_specs=[
            pl.BlockSpec((n, n), lambda i: (0, 0)),
            pl.BlockSpec((n, h, b), lambda i: (0, 0, 0)),
            pl.BlockSpec((nb_g, 2 * ci, b), lambda i: (i, 0, 0)),
            pl.BlockSpec((nb_g, kc, 2 * h), lambda i: (i, 0, 0)),
        ],
        out_specs=(pl.BlockSpec((nb_g, h, b), lambda i: (i, 0, 0)),
                   pl.BlockSpec((nb_g, h, b), lambda i: (i, 0, 0))),
        scratch_shapes=[pltpu.VMEM((n, h, b), bf16)],
        compiler_params=cparams,
    )(a_adj, s_t, xa3, wg3)

    h3 = pl.pallas_call(
        _candagg_kernel,
        grid=(n // nb_g,),
        out_shape=jax.ShapeDtypeStruct((n, h, b), bf16),
        in_specs=[
            pl.BlockSpec((n, n), lambda i: (0, 0)),
            pl.BlockSpec((n, h, b), lambda i: (0, 0, 0)),
            pl.BlockSpec((nb_g, 2 * ci, b), lambda i: (i, 0, 0)),
            pl.BlockSpec((nb_g, kc, h), lambda i: (i, 0, 0)),
            pl.BlockSpec((nb_g, h, b), lambda i: (i, 0, 0)),
            pl.BlockSpec((nb_g, h, b), lambda i: (i, 0, 0)),
        ],
        out_specs=pl.BlockSpec((nb_g, h, b), lambda i: (i, 0, 0)),
        scratch_shapes=[pltpu.VMEM((n, h, b), bf16)],
        compiler_params=cparams,
    )(a_adj, t3, xa3, wu3, r3, s_t)

    return h3.transpose(2, 0, 1).astype(out_dtype)


# speedup vs baseline: 1.6072x; 1.0388x over previous
import jax
import jax.numpy as jnp
from jax.experimental import pallas as pl
from jax.experimental.pallas import tpu as pltpu

f32 = jnp.float32
bf16 = jnp.bfloat16


def _prep_kernel(eb_ref, e_ref, x_ref,
                 gcat_ref, ucat_ref,
                 a_ref, xa_ref, wg_ref, wu_ref):
    eb = eb_ref[...]
    e = e_ref[...]
    g = jax.lax.dot_general(eb, e, (((1,), (1,)), ((), ())),
                            preferred_element_type=f32)
    g = jnp.maximum(g, 0.0)
    g = g - jnp.max(g, axis=1, keepdims=True)
    eg = jnp.exp(g)
    a = (eg / jnp.sum(eg, axis=1, keepdims=True)).astype(bf16)
    a_ref[...] = a
    nn, ci, bb = x_ref.shape
    x2 = x_ref[...].reshape(nn, ci * bb)
    aggx = jnp.dot(a, x2, preferred_element_type=f32).astype(bf16)
    xa_ref[:, ci:, :] = aggx.reshape(a.shape[0], ci, bb)
    nbw = a.shape[0]
    base = pl.program_id(0) * nbw
    xa_ref[:, :ci, :] = x_ref[pl.ds(base, nbw)]
    nb, kc, og = wg_ref.shape
    ou = wu_ref.shape[2]
    wg = jnp.dot(eb, gcat_ref[...], preferred_element_type=f32).astype(bf16)
    wu = jnp.dot(eb, ucat_ref[...], preferred_element_type=f32).astype(bf16)
    wg_ref[...] = wg.reshape(nb, kc, og)
    wu_ref[...] = wu.reshape(nb, kc, ou)


def _fill_agg(a_ref, v_ref, agg_ref):
    nn, hh, bb = v_ref.shape
    a = a_ref[...]
    csz = 8 if hh % 8 == 0 else hh
    for c in range(0, hh, csz):
        vc = v_ref[:, c:c + csz, :].reshape(nn, csz * bb)
        chunk = jnp.dot(a, vc, preferred_element_type=f32).astype(bf16)
        agg_ref[:, c:c + csz, :] = chunk.reshape(nn, csz, bb)


def _gateagg_kernel(a_ref, sf_ref, xa_ref, wg_ref,
                    t_ref, r_ref, aggs_ref):
    nb = wg_ref.shape[0]
    i = pl.program_id(0)

    @pl.when(i == 0)
    def _():
        _fill_agg(a_ref, sf_ref, aggs_ref)

    ones = jnp.ones((1, sf_ref.shape[2]), bf16)
    for j in range(nb):
        jj = i * nb + j
        s = sf_ref[jj]
        cat = jnp.concatenate(
            [s, aggs_ref[jj], xa_ref[j], ones], axis=0)
        pre = jax.lax.dot_general(wg_ref[j], cat, (((0,), (0,)), ((), ())),
                                  preferred_element_type=f32)
        zr = jax.nn.sigmoid(pre)
        h = s.shape[0]
        z = zr[:h, :]
        t_ref[j] = (z * s.astype(f32)).astype(bf16)
        r_ref[j] = zr[h:, :].astype(bf16)


def _candagg_kernel(a_ref, tf_ref, xa_ref, wu_ref,
                    r_ref, s_ref, h_ref, aggt_ref):
    nb = wu_ref.shape[0]
    i = pl.program_id(0)

    @pl.when(i == 0)
    def _():
        _fill_agg(a_ref, tf_ref, aggt_ref)

    ones = jnp.ones((1, tf_ref.shape[2]), bf16)
    for j in range(nb):
        jj = i * nb + j
        cat = jnp.concatenate(
            [tf_ref[jj], aggt_ref[jj], xa_ref[j], ones], axis=0)
        pre = jax.lax.dot_general(wu_ref[j], cat, (((0,), (0,)), ((), ())),
                                  preferred_element_type=f32)
        hc = jnp.tanh(pre)
        r = r_ref[j].astype(f32)
        hv = r * s_ref[j].astype(f32) + (1.0 - r) * hc
        h_ref[j] = hv.astype(bf16)


def kernel(x, state, node_emb, gate_w, gate_b, upd_w, upd_b):
    b, n, ci = x.shape
    h = state.shape[-1]
    d = node_emb.shape[-1]
    out_dtype = state.dtype
    kc = 2 * h + 2 * ci + 1

    e = node_emb.astype(f32)
    x_t = x.astype(bf16).transpose(1, 2, 0)
    s_t = state.astype(bf16).transpose(1, 2, 0)

    gw = gate_w.astype(f32)
    uw = upd_w.astype(f32)
    gcat = jnp.concatenate(
        [gw[:, 0, ci:, :], gw[:, 1, ci:, :],
         gw[:, 0, :ci, :], gw[:, 1, :ci, :],
         gate_b.astype(f32)[:, None, :]], axis=1).reshape(d, kc * 2 * h)
    ucat = jnp.concatenate(
        [uw[:, 0, ci:, :], uw[:, 1, ci:, :],
         uw[:, 0, :ci, :], uw[:, 1, :ci, :],
         upd_b.astype(f32)[:, None, :]], axis=1).reshape(d, kc * h)

    nb_w = 64 if n % 64 == 0 else n
    a_adj, xa3, wg3, wu3 = pl.pallas_call(
        _prep_kernel,
        grid=(n // nb_w,),
        out_shape=(jax.ShapeDtypeStruct((n, n), bf16),
                   jax.ShapeDtypeStruct((n, 2 * ci, b), bf16),
                   jax.ShapeDtypeStruct((n, kc, 2 * h), bf16),
                   jax.ShapeDtypeStruct((n, kc, h), bf16)),
        in_specs=[
            pl.BlockSpec((nb_w, d), lambda i: (i, 0)),
            pl.BlockSpec((n, d), lambda i: (0, 0)),
            pl.BlockSpec((n, ci, b), lambda i: (0, 0, 0)),
            pl.BlockSpec((d, kc * 2 * h), lambda i: (0, 0)),
            pl.BlockSpec((d, kc * h), lambda i: (0, 0)),
        ],
        out_specs=(pl.BlockSpec((nb_w, n), lambda i: (i, 0)),
                   pl.BlockSpec((nb_w, 2 * ci, b), lambda i: (i, 0, 0)),
                   pl.BlockSpec((nb_w, kc, 2 * h), lambda i: (i, 0, 0)),
                   pl.BlockSpec((nb_w, kc, h), lambda i: (i, 0, 0))),
        compiler_params=pltpu.CompilerParams(
            dimension_semantics=("parallel",)),
    )(e, e, x_t, gcat, ucat)

    nb_g = 64 if n % 64 == 0 else n
    cparams = pltpu.CompilerParams(
        dimension_semantics=("arbitrary",),
        vmem_limit_bytes=56 * 1024 * 1024)
    t3, r3 = pl.pallas_call(
        _gateagg_kernel,
        grid=(n // nb_g,),
        out_shape=(jax.ShapeDtypeStruct((n, h, b), bf16),
                   jax.ShapeDtypeStruct((n, h, b), bf16)),
        in_specs=[
            pl.BlockSpec((n, n), lambda i: (0, 0)),
            pl.BlockSpec((n, h, b), lambda i: (0, 0, 0)),
            pl.BlockSpec((nb_g, 2 * ci, b), lambda i: (i, 0, 0)),
            pl.BlockSpec((nb_g, kc, 2 * h), lambda i: (i, 0, 0)),
        ],
        out_specs=(pl.BlockSpec((nb_g, h, b), lambda i: (i, 0, 0)),
                   pl.BlockSpec((nb_g, h, b), lambda i: (i, 0, 0))),
        scratch_shapes=[pltpu.VMEM((n, h, b), bf16)],
        compiler_params=cparams,
    )(a_adj, s_t, xa3, wg3)

    h3 = pl.pallas_call(
        _candagg_kernel,
        grid=(n // nb_g,),
        out_shape=jax.ShapeDtypeStruct((n, h, b), bf16),
        in_specs=[
            pl.BlockSpec((n, n), lambda i: (0, 0)),
            pl.BlockSpec((n, h, b), lambda i: (0, 0, 0)),
            pl.BlockSpec((nb_g, 2 * ci, b), lambda i: (i, 0, 0)),
            pl.BlockSpec((nb_g, kc, h), lambda i: (i, 0, 0)),
            pl.BlockSpec((nb_g, h, b), lambda i: (i, 0, 0)),
            pl.BlockSpec((nb_g, h, b), lambda i: (i, 0, 0)),
        ],
        out_specs=pl.BlockSpec((nb_g, h, b), lambda i: (i, 0, 0)),
        scratch_shapes=[pltpu.VMEM((n, h, b), bf16)],
        compiler_params=cparams,
    )(a_adj, t3, xa3, wu3, r3, s_t)

    return h3.transpose(2, 0, 1).astype(out_dtype)
